# baseline (speedup 1.0000x reference)
import functools

import jax
import jax.numpy as jnp
from jax import lax
from jax.experimental import pallas as pl
from jax.experimental.pallas import tpu as pltpu
from jax.experimental.pallas import tpu_sc as plsc

_TILE = 512
_NSTREAM = 4
_STEP = _TILE * _NSTREAM

_NW = 32
_LANES = 16


def _probs_tile(x_tile, w, bias):
    scores = jax.lax.dot_general(
        x_tile, w,
        (((1,), (1,)), ((), ())),
        preferred_element_type=jnp.float32,
    )
    scores = scores + bias
    m = jnp.max(scores, axis=-1, keepdims=True)
    e = jnp.exp(scores - m)
    s = e / jnp.sum(e, axis=-1, keepdims=True)
    return s.T


def _probs_body(*refs):
    x_refs = refs[:_NSTREAM]
    w_ref, b_ref = refs[_NSTREAM], refs[_NSTREAM + 1]
    s_out_ref = refs[_NSTREAM + 2]
    w = w_ref[...]
    bias = b_ref[...]
    for k in range(_NSTREAM):
        s_out_ref[:, pl.ds(k * _TILE, _TILE)] = _probs_tile(
            x_refs[k][...], w, bias)


def _tc_probs(x, W, b):
    ntok, dim = x.shape
    nexp = W.shape[0]
    return pl.pallas_call(
        _probs_body,
        grid=(ntok // _STEP,),
        in_specs=[
            pl.BlockSpec((_TILE, dim), (lambda i, k=k: (i * _NSTREAM + k, 0)))
            for k in range(_NSTREAM)
        ] + [
            pl.BlockSpec((nexp, dim), lambda i: (0, 0)),
            pl.BlockSpec((nexp,), lambda i: (0,)),
        ],
        out_specs=pl.BlockSpec((nexp, _STEP), lambda i: (0, i)),
        out_shape=jax.ShapeDtypeStruct((nexp, ntok), jnp.float32),
        compiler_params=pltpu.CompilerParams(
            dimension_semantics=("parallel",),
        ),
    )(*([x] * _NSTREAM), W, b)


def _make_sc_top2(ntok, nexp):
    per_w = ntok // _NW
    ngroups = per_w // _LANES
    mesh = plsc.VectorSubcoreMesh(core_axis_name="c", subcore_axis_name="s")

    @functools.partial(
        pl.kernel,
        mesh=mesh,
        out_type=[
            jax.ShapeDtypeStruct((2, ntok), jnp.float32),
            jax.ShapeDtypeStruct((2, ntok), jnp.int32),
        ],
        scratch_types=[
            pltpu.VMEM((nexp, per_w), jnp.float32),
            pltpu.VMEM((2, per_w), jnp.float32),
            pltpu.VMEM((2, per_w), jnp.int32),
        ],
    )
    def sc_top2(s_hbm, w_hbm, i_hbm, s_v, w_v, i_v):
        wid = lax.axis_index("s") * 2 + lax.axis_index("c")
        base = wid * per_w
        pltpu.sync_copy(s_hbm.at[:, pl.ds(base, per_w)], s_v)

        def group(g, carry):
            off = g * _LANES
            m1 = jnp.full((_LANES,), -jnp.inf, jnp.float32)
            m2 = jnp.full((_LANES,), -jnp.inf, jnp.float32)
            i1 = jnp.zeros((_LANES,), jnp.int32)
            i2 = jnp.zeros((_LANES,), jnp.int32)
            for e in range(nexp):
                v = s_v[e, pl.ds(off, _LANES)]
                ev = jnp.full((_LANES,), e, jnp.int32)
                best = v > m1
                beats2 = v > m2
                i2 = jnp.where(best, i1, jnp.where(beats2, ev, i2))
                m2 = jnp.where(best, m1, jnp.where(beats2, v, m2))
                i1 = jnp.where(best, ev, i1)
                m1 = jnp.where(best, v, m1)
            w_v[0, pl.ds(off, _LANES)] = m1
            w_v[1, pl.ds(off, _LANES)] = m2
            i_v[0, pl.ds(off, _LANES)] = i1
            i_v[1, pl.ds(off, _LANES)] = i2
            return carry

        lax.fori_loop(0, ngroups, group, 0)
        pltpu.sync_copy(w_v, w_hbm.at[:, pl.ds(base, per_w)])
        pltpu.sync_copy(i_v, i_hbm.at[:, pl.ds(base, per_w)])

    return sc_top2


@jax.jit
def kernel(x, W, b):
    ntok, _ = x.shape
    nexp = W.shape[0]
    s_t = _tc_probs(x, W, b)
    w_t, i_t = _make_sc_top2(ntok, nexp)(s_t)
    return w_t.T, i_t.T


# speedup vs baseline: 1.4103x; 1.4103x over previous
import jax
import jax.numpy as jnp
from jax.experimental import pallas as pl
from jax.experimental.pallas import tpu as pltpu

_TILE = 256
_NSTREAM = 8
_STEP = _TILE * _NSTREAM


def _route_tile(x_tile, w, bias):
    scores = jax.lax.dot_general(
        x_tile, w,
        (((1,), (1,)), ((), ())),
        preferred_element_type=jnp.float32,
    )
    scores = scores + bias
    m = jnp.max(scores, axis=-1, keepdims=True)
    e = jnp.exp(scores - m)
    s = e / jnp.sum(e, axis=-1, keepdims=True)
    n = s.shape[-1]
    iota = jax.lax.broadcasted_iota(jnp.int32, s.shape, 1)
    m1 = jnp.max(s, axis=-1, keepdims=True)
    i1 = jnp.min(jnp.where(s == m1, iota, n), axis=-1, keepdims=True)
    s2 = jnp.where(iota == i1, -jnp.inf, s)
    m2 = jnp.max(s2, axis=-1, keepdims=True)
    i2 = jnp.min(jnp.where(s2 == m2, iota, n), axis=-1, keepdims=True)
    w2 = jnp.concatenate([m1, m2], axis=1)
    i2c = jnp.concatenate([i1, i2], axis=1)
    return w2.T, i2c.T


def _router_body(*refs):
    x_refs = refs[:_NSTREAM]
    w_ref, b_ref = refs[_NSTREAM], refs[_NSTREAM + 1]
    w_out_ref, i_out_ref = refs[_NSTREAM + 2], refs[_NSTREAM + 3]
    w = w_ref[...]
    bias = b_ref[...]
    for k in range(0, _NSTREAM, 2):
        x_pair = jnp.concatenate([x_refs[k][...], x_refs[k + 1][...]], axis=0)
        wk, ik = _route_tile(x_pair, w, bias)
        w_out_ref[:, pl.ds(k * _TILE, 2 * _TILE)] = wk
        i_out_ref[:, pl.ds(k * _TILE, 2 * _TILE)] = ik


@jax.jit
def kernel(x, W, b):
    ntok, dim = x.shape
    nexp = W.shape[0]
    grid = (ntok // _STEP,)

    weights_t, idx_t = pl.pallas_call(
        _router_body,
        grid=grid,
        in_specs=[
            pl.BlockSpec((_TILE, dim), (lambda i, k=k: (i * _NSTREAM + k, 0)))
            for k in range(_NSTREAM)
        ] + [
            pl.BlockSpec((nexp, dim), lambda i: (0, 0)),
            pl.BlockSpec((nexp,), lambda i: (0,)),
        ],
        out_specs=[
            pl.BlockSpec((2, _STEP), lambda i: (0, i)),
            pl.BlockSpec((2, _STEP), lambda i: (0, i)),
        ],
        out_shape=[
            jax.ShapeDtypeStruct((2, ntok), jnp.float32),
            jax.ShapeDtypeStruct((2, ntok), jnp.int32),
        ],
        compiler_params=pltpu.CompilerParams(
            dimension_semantics=("parallel",),
        ),
    )(*([x] * _NSTREAM), W, b)
    return weights_t.T, idx_t.T


# speedup vs baseline: 1.4416x; 1.0222x over previous
import jax
import jax.numpy as jnp
from jax.experimental import pallas as pl
from jax.experimental.pallas import tpu as pltpu

_TILE = 128
_NSTREAM = 16
_STEP = _TILE * _NSTREAM


def _route_tile(x_tile, w, bias):
    scores = jax.lax.dot_general(
        x_tile, w,
        (((1,), (1,)), ((), ())),
        preferred_element_type=jnp.float32,
    )
    scores = scores + bias
    m = jnp.max(scores, axis=-1, keepdims=True)
    e = jnp.exp(scores - m)
    s = e / jnp.sum(e, axis=-1, keepdims=True)
    n = s.shape[-1]
    iota = jax.lax.broadcasted_iota(jnp.int32, s.shape, 1)
    m1 = jnp.max(s, axis=-1, keepdims=True)
    i1 = jnp.min(jnp.where(s == m1, iota, n), axis=-1, keepdims=True)
    s2 = jnp.where(iota == i1, -jnp.inf, s)
    m2 = jnp.max(s2, axis=-1, keepdims=True)
    i2 = jnp.min(jnp.where(s2 == m2, iota, n), axis=-1, keepdims=True)
    w2 = jnp.concatenate([m1, m2], axis=1)
    i2c = jnp.concatenate([i1, i2], axis=1)
    return w2.T, i2c.T


def _router_body(*refs):
    x_refs = refs[:_NSTREAM]
    w_ref, b_ref = refs[_NSTREAM], refs[_NSTREAM + 1]
    w_out_ref, i_out_ref = refs[_NSTREAM + 2], refs[_NSTREAM + 3]
    w = w_ref[...]
    bias = b_ref[...]
    for k in range(0, _NSTREAM, 4):
        x_pair = jnp.concatenate([r[...] for r in x_refs[k:k + 4]], axis=0)
        wk, ik = _route_tile(x_pair, w, bias)
        w_out_ref[:, pl.ds(k * _TILE, 4 * _TILE)] = wk
        i_out_ref[:, pl.ds(k * _TILE, 4 * _TILE)] = ik


@jax.jit
def kernel(x, W, b):
    ntok, dim = x.shape
    nexp = W.shape[0]
    grid = (ntok // _STEP,)

    weights_t, idx_t = pl.pallas_call(
        _router_body,
        grid=grid,
        in_specs=[
            pl.BlockSpec((_TILE, dim), (lambda i, k=k: (i * _NSTREAM + k, 0)))
            for k in range(_NSTREAM)
        ] + [
            pl.BlockSpec((nexp, dim), lambda i: (0, 0)),
            pl.BlockSpec((nexp,), lambda i: (0,)),
        ],
        out_specs=[
            pl.BlockSpec((2, _STEP), lambda i: (0, i)),
            pl.BlockSpec((2, _STEP), lambda i: (0, i)),
        ],
        out_shape=[
            jax.ShapeDtypeStruct((2, ntok), jnp.float32),
            jax.ShapeDtypeStruct((2, ntok), jnp.int32),
        ],
        compiler_params=pltpu.CompilerParams(
            dimension_semantics=("parallel",),
        ),
    )(*([x] * _NSTREAM), W, b)
    return weights_t.T, idx_t.T
